# Initial kernel scaffold; baseline (speedup 1.0000x reference)
import jax
import jax.numpy as jnp
from jax.experimental import pallas as pl


def kernel(x, pk_embeddings, pk_predictions, edge_index, Wg, bg, W1, b1, W2, b2, W3, b3, ln1_w, ln1_b, ln2_w, ln2_b, ln3_w, ln3_b, Wp1, bp1, Wp2, bp2, Wr1, br1, Wr2, br2, rw):
    raise NotImplementedError("write your pallas kernel here")


# speedup vs baseline: 10.3838x; 10.3838x over previous
import functools

import jax
import jax.numpy as jnp
from jax import lax
from jax.experimental import pallas as pl
from jax.experimental.pallas import tpu as pltpu
from jax.experimental.pallas import tpu_sc as plsc

N = 100000
F = 48
FB = 16
NC = 2
NS = 16
NW = NC * NS
ACC_ROWS = 102400
RPS = ACC_ROWS // NS
KW = 8
WEDGE = KW * 128
BR = 400
GRID = N // BR
EPS = 1e-5
HIGH = lax.Precision.DEFAULT

_mesh = functools.partial(plsc.VectorSubcoreMesh,
                          core_axis_name="c", subcore_axis_name="s")
_sc_params = pltpu.CompilerParams(use_tc_tiling_on_sc=False)


def _zero_slice(zeros_ref, acc_ref, base):
    zn = zeros_ref.shape[0]
    full, rem = RPS // zn, RPS % zn
    for k in range(full):
        pltpu.sync_copy(zeros_ref, acc_ref.at[pl.ds(base + k * zn, zn)])
    if rem:
        pltpu.sync_copy(zeros_ref.at[pl.ds(0, rem)],
                        acc_ref.at[pl.ds(base + full * zn, rem)])


def _sc_degree(dst2d, nwin):
    rows_per_worker = nwin * KW

    def body(dst_hbm, out_hbm, ones_v, zeros_v, dst_w, acc):
        c = lax.axis_index("c")
        s = lax.axis_index("s")
        w = c * NS + s
        for i in range(8):
            ones_v[pl.ds(i * 16, 16)] = jnp.ones((16,), jnp.float32)
        for i in range(32):
            zeros_v[pl.ds(i * 16, 16)] = jnp.zeros((16,), jnp.float32)
        _zero_slice(zeros_v, acc, s * RPS)
        plsc.subcore_barrier()

        def win(t, carry):
            rowbase = w * rows_per_worker + t * KW
            pltpu.sync_copy(dst_hbm.at[pl.ds(rowbase, KW)], dst_w)
            for j in range(KW):
                pltpu.sync_copy(ones_v, acc.at[dst_w.at[j]], add=True)
            return carry

        lax.fori_loop(0, nwin, win, 0)
        plsc.subcore_barrier()
        pltpu.sync_copy(acc.at[pl.ds(s * RPS, RPS)], out_hbm.at[c, s])

    k = pl.kernel(
        body,
        out_type=jax.ShapeDtypeStruct((NC, NS, RPS), jnp.float32),
        mesh=_mesh(),
        compiler_params=_sc_params,
        scratch_types=[
            pltpu.VMEM((128,), jnp.float32),
            pltpu.VMEM((512,), jnp.float32),
            pltpu.VMEM((KW, 128), jnp.int32),
            pltpu.VMEM_SHARED((ACC_ROWS,), jnp.float32),
        ],
    )
    return k(dst2d)


def _sc_scatter(g0, g1, g2, src2d, dst2d, nwin):
    rows_per_worker = nwin * KW

    def body(g0_hbm, g1_hbm, g2_hbm, src_hbm, dst_hbm,
             o0, o1, o2, zeros_z, src_w, dst_w, rows_v, acc, sem):
        c = lax.axis_index("c")
        s = lax.axis_index("s")
        w = c * NS + s

        def zb(i, carry):
            zeros_z[i, :] = jnp.zeros((16,), jnp.float32)
            return carry

        lax.fori_loop(0, 256, zb, 0)

        for g_hbm, o_hbm in ((g0_hbm, o0), (g1_hbm, o1), (g2_hbm, o2)):
            _zero_slice(zeros_z, acc, s * RPS)
            plsc.subcore_barrier()

            def win(t, carry, g_hbm=g_hbm):
                rowbase = w * rows_per_worker + t * KW
                pltpu.sync_copy(src_hbm.at[pl.ds(rowbase, KW)], src_w)
                pltpu.sync_copy(dst_hbm.at[pl.ds(rowbase, KW)], dst_w)
                descs = [
                    pltpu.async_copy(g_hbm.at[src_w.at[j]],
                                     rows_v.at[pl.ds(j * 128, 128)], sem)
                    for j in range(KW)
                ]
                for d in descs:
                    d.wait()
                descs = [
                    pltpu.async_copy(rows_v.at[pl.ds(j * 128, 128)],
                                     acc.at[dst_w.at[j]], sem, add=True)
                    for j in range(KW)
                ]
                for d in descs:
                    d.wait()
                return carry

            lax.fori_loop(0, nwin, win, 0)
            plsc.subcore_barrier()
            pltpu.sync_copy(acc.at[pl.ds(s * RPS, RPS)], o_hbm.at[c, s])

    out_sd = jax.ShapeDtypeStruct((NC, NS, RPS, FB), jnp.float32)
    k = pl.kernel(
        body,
        out_type=(out_sd, out_sd, out_sd),
        mesh=_mesh(),
        compiler_params=_sc_params,
        scratch_types=[
            pltpu.VMEM((256, FB), jnp.float32),
            pltpu.VMEM((KW, 128), jnp.int32),
            pltpu.VMEM((KW, 128), jnp.int32),
            pltpu.VMEM((WEDGE, FB), jnp.float32),
            pltpu.VMEM_SHARED((ACC_ROWS, FB), jnp.float32),
            pltpu.SemaphoreType.DMA,
        ],
    )
    return k(g0, g1, g2, src2d, dst2d)


def _row_spec(width):
    return pl.BlockSpec((BR, width), lambda i: (i, 0))


def _full_spec(shape):
    nd = len(shape)
    return pl.BlockSpec(shape, lambda i, nd=nd: (0,) * nd)


_degp_spec = pl.BlockSpec((NC, BR, 1), lambda i: (0, i, 0))
_part_spec = pl.BlockSpec((NC, BR, FB), lambda i: (0, i, 0))
_smem_spec = pl.BlockSpec(memory_space=pltpu.SMEM)


def _dinv(degp_blk):
    return lax.rsqrt(1.0 + degp_blk[0] + degp_blk[1])


def _prep_body(x_r, pk_r, pkp_r, degp_r, wgt_r, bg_r, w1t_r,
               comb_o, gate_o, g0_o, g1_o, g2_o):
    comb = jnp.concatenate([x_r[...], pk_r[...], pkp_r[...]], axis=-1)
    comb_o[...] = comb
    gate_o[...] = jax.nn.sigmoid(
        jnp.dot(comb, wgt_r[...], precision=HIGH) + bg_r[...])
    dinv = _dinv(degp_r[...])
    g = jnp.dot(comb, w1t_r[...], precision=HIGH) * dinv
    g0_o[...] = g[:, 0:16]
    g1_o[...] = g[:, 16:32]
    g2_o[...] = g[:, 32:48]


def _tc_prep(x, pk, pkp, degp, WgT, bg2, W1T):
    return pl.pallas_call(
        _prep_body,
        grid=(GRID,),
        in_specs=[_row_spec(32), _row_spec(15), _row_spec(1), _degp_spec,
                  _full_spec((F, F)), _full_spec((1, F)), _full_spec((F, F))],
        out_specs=(_row_spec(F), _row_spec(F),
                   _row_spec(FB), _row_spec(FB), _row_spec(FB)),
        out_shape=(jax.ShapeDtypeStruct((N, F), jnp.float32),
                   jax.ShapeDtypeStruct((N, F), jnp.float32),
                   jax.ShapeDtypeStruct((N, FB), jnp.float32),
                   jax.ShapeDtypeStruct((N, FB), jnp.float32),
                   jax.ShapeDtypeStruct((N, FB), jnp.float32)),
    )(x, pk, pkp, degp, WgT, bg2, W1T)


def _post_body(p0_r, p1_r, p2_r, g0_r, g1_r, g2_r, degp_r, b_r, t_o, st_o):
    i = pl.program_id(0)
    dinv = _dinv(degp_r[...])
    cols = []
    for pf, gf in ((p0_r, g0_r), (p1_r, g1_r), (p2_r, g2_r)):
        a = pf[...]
        cols.append(a[0] + a[1] + gf[...])
    t = jnp.concatenate(cols, axis=-1) * dinv + b_r[...]
    t_o[...] = t

    @pl.when(i == 0)
    def _():
        st_o[0, 0] = 0.0
        st_o[0, 1] = 0.0

    st_o[0, 0] += jnp.sum(t)
    st_o[0, 1] += jnp.sum(t * t)


def _tc_post(p0, p1, p2, g0, g1, g2, degp, b2):
    return pl.pallas_call(
        _post_body,
        grid=(GRID,),
        in_specs=[_part_spec, _part_spec, _part_spec,
                  _row_spec(FB), _row_spec(FB), _row_spec(FB),
                  _degp_spec, _full_spec((1, F))],
        out_specs=(_row_spec(F), _smem_spec),
        out_shape=(jax.ShapeDtypeStruct((N, F), jnp.float32),
                   jax.ShapeDtypeStruct((1, 2), jnp.float32)),
    )(p0, p1, p2, g0, g1, g2, degp, b2)


def _apply_body(gated, with_next, t_r, st_r, lw_r, lb_r, aux_r, degp_r,
                *rest):
    inv_cnt = 1.0 / (N * F)
    mean = st_r[0, 0] * inv_cnt
    var = st_r[0, 1] * inv_cnt - mean * mean
    std = jnp.sqrt(jnp.maximum(var, 0.0))
    tn = (t_r[...] - mean) / (std + EPS) * lw_r[...] + lb_r[...]
    r = jnp.maximum(tn, 0.0)
    h = r * aux_r[...] if gated else aux_r[...] + r
    if with_next:
        wnt_r, h_o, g0_o, g1_o, g2_o = rest
        h_o[...] = h
        dinv = _dinv(degp_r[...])
        g = jnp.dot(h, wnt_r[...], precision=HIGH) * dinv
        g0_o[...] = g[:, 0:16]
        g1_o[...] = g[:, 16:32]
        g2_o[...] = g[:, 32:48]
    else:
        (h_o,) = rest
        h_o[...] = h


def _tc_apply(t, stats, lw2, lb2, aux, degp, WnT, gated):
    with_next = WnT is not None
    in_specs = [_row_spec(F), _smem_spec, _full_spec((1, F)),
                _full_spec((1, F)), _row_spec(F), _degp_spec]
    args = [t, stats, lw2, lb2, aux, degp]
    out_specs = [_row_spec(F)]
    out_shape = [jax.ShapeDtypeStruct((N, F), jnp.float32)]
    if with_next:
        in_specs.append(_full_spec((F, F)))
        args.append(WnT)
        out_specs += [_row_spec(FB)] * 3
        out_shape += [jax.ShapeDtypeStruct((N, FB), jnp.float32)] * 3
    return pl.pallas_call(
        functools.partial(_apply_body, gated, with_next),
        grid=(GRID,),
        in_specs=in_specs,
        out_specs=tuple(out_specs),
        out_shape=tuple(out_shape),
    )(*args)


def _heads_body(h_r, c_r, wp1_r, bp1_r, wp2_r, bp2_r,
                wr1_r, br1_r, wr2_r, br2_r, rw_r, o_r):
    pm = jnp.dot(jnp.maximum(
        jnp.dot(h_r[...], wp1_r[...], precision=HIGH) + bp1_r[...], 0.0),
        wp2_r[...], precision=HIGH) + bp2_r[0, 0]
    pr = jnp.dot(jnp.maximum(
        jnp.dot(c_r[...], wr1_r[...], precision=HIGH) + br1_r[...], 0.0),
        wr2_r[...], precision=HIGH) + br2_r[0, 0]
    o_r[...] = pm + rw_r[0, 0] * pr


def _tc_heads(h, comb, Wp1T, bp12, Wp2T, bp2s, Wr1T, br12, Wr2T, br2s, rws):
    return pl.pallas_call(
        _heads_body,
        grid=(GRID,),
        in_specs=[_row_spec(F), _row_spec(F),
                  _full_spec((F, 24)), _full_spec((1, 24)),
                  _full_spec((24, 1)), _smem_spec,
                  _full_spec((F, 24)), _full_spec((1, 24)),
                  _full_spec((24, 1)), _smem_spec, _smem_spec],
        out_specs=_row_spec(1),
        out_shape=jax.ShapeDtypeStruct((N, 1), jnp.float32),
    )(h, comb, Wp1T, bp12, Wp2T, bp2s, Wr1T, br12, Wr2T, br2s, rws)


def kernel(x, pk_embeddings, pk_predictions, edge_index, Wg, bg, W1, b1,
           W2, b2, W3, b3, ln1_w, ln1_b, ln2_w, ln2_b, ln3_w, ln3_b,
           Wp1, bp1, Wp2, bp2, Wr1, br1, Wr2, br2, rw):
    E = edge_index.shape[1]
    nwin = -(-E // (NW * WEDGE))
    pad = NW * WEDGE * nwin - E
    pad_idx = jnp.arange(pad, dtype=jnp.int32) % 128
    src2d = jnp.concatenate([edge_index[0], pad_idx]).reshape(-1, 128)
    dst2d = jnp.concatenate([edge_index[1], N + pad_idx]).reshape(-1, 128)

    degp = _sc_degree(dst2d, nwin).reshape(NC, ACC_ROWS, 1)

    comb, gate, g0, g1, g2 = _tc_prep(
        x, pk_embeddings, pk_predictions, degp,
        Wg.T, bg.reshape(1, F), W1.T)

    layers = [(b1, ln1_w, ln1_b, W2), (b2, ln2_w, ln2_b, W3),
              (b3, ln3_w, ln3_b, None)]
    aux = gate
    h = None
    for li, (bi, lw, lb, Wn) in enumerate(layers):
        o0, o1, o2 = _sc_scatter(g0, g1, g2, src2d, dst2d, nwin)
        p0 = o0.reshape(NC, ACC_ROWS, FB)
        p1 = o1.reshape(NC, ACC_ROWS, FB)
        p2 = o2.reshape(NC, ACC_ROWS, FB)
        t, stats = _tc_post(p0, p1, p2, g0, g1, g2, degp, bi.reshape(1, F))
        res = _tc_apply(t, stats, lw.reshape(1, F), lb.reshape(1, F),
                        aux, degp, None if Wn is None else Wn.T,
                        gated=(li == 0))
        if Wn is None:
            h = res[0]
        else:
            h, g0, g1, g2 = res
        aux = h

    return _tc_heads(h, comb, Wp1.T, bp1.reshape(1, 24), Wp2.T,
                     bp2.reshape(1, 1), Wr1.T, br1.reshape(1, 24), Wr2.T,
                     br2.reshape(1, 1), rw.reshape(1, 1))


# speedup vs baseline: 11.9816x; 1.1539x over previous
import functools

import jax
import jax.numpy as jnp
from jax import lax
from jax.experimental import pallas as pl
from jax.experimental.pallas import tpu as pltpu
from jax.experimental.pallas import tpu_sc as plsc

N = 100000
F = 48
FB = 16
NC = 2
NS = 16
NW = NC * NS
NPAD = 102400
RPS = NPAD // NS
KW = 4
WEDGE = KW * 128
BC = 2048
GRID = NPAD // BC
EPS = 1e-5
PREC = lax.Precision.DEFAULT

_mesh = functools.partial(plsc.VectorSubcoreMesh,
                          core_axis_name="c", subcore_axis_name="s")
_sc_params = pltpu.CompilerParams(use_tc_tiling_on_sc=False)


def _zero_slice(zeros_ref, acc_ref, base):
    zn = zeros_ref.shape[0]
    full, rem = RPS // zn, RPS % zn
    for k in range(full):
        pltpu.sync_copy(zeros_ref, acc_ref.at[pl.ds(base + k * zn, zn)])
    if rem:
        pltpu.sync_copy(zeros_ref.at[pl.ds(0, rem)],
                        acc_ref.at[pl.ds(base + full * zn, rem)])


def _sc_degree(dst2d, nwin):
    rows_per_worker = nwin * KW

    def body(dst_hbm, out_hbm, ones_v, zeros_v, dst_w, acc):
        c = lax.axis_index("c")
        s = lax.axis_index("s")
        w = c * NS + s
        for i in range(8):
            ones_v[pl.ds(i * 16, 16)] = jnp.ones((16,), jnp.float32)
        for i in range(32):
            zeros_v[pl.ds(i * 16, 16)] = jnp.zeros((16,), jnp.float32)
        _zero_slice(zeros_v, acc, s * RPS)
        plsc.subcore_barrier()

        def win(t, carry):
            rowbase = w * rows_per_worker + t * KW
            pltpu.sync_copy(dst_hbm.at[pl.ds(rowbase, KW)], dst_w)
            for j in range(KW):
                pltpu.sync_copy(ones_v, acc.at[dst_w.at[j]], add=True)
            return carry

        lax.fori_loop(0, nwin, win, 0)
        plsc.subcore_barrier()
        pltpu.sync_copy(acc.at[pl.ds(s * RPS, RPS)], out_hbm.at[c, s])

    k = pl.kernel(
        body,
        out_type=jax.ShapeDtypeStruct((NC, NS, RPS), jnp.float32),
        mesh=_mesh(),
        compiler_params=_sc_params,
        scratch_types=[
            pltpu.VMEM((128,), jnp.float32),
            pltpu.VMEM((512,), jnp.float32),
            pltpu.VMEM((KW, 128), jnp.int32),
            pltpu.VMEM_SHARED((NPAD,), jnp.float32),
        ],
    )
    return k(dst2d)


def _sc_scatter(g0, g1, g2, src2d, dst2d, nwin):
    rows_per_worker = nwin * KW

    def body(g0_hbm, g1_hbm, g2_hbm, src_hbm, dst_hbm,
             out_hbm, zeros_z, src_w, dst_w, rows_v, acc, sem):
        c = lax.axis_index("c")
        s = lax.axis_index("s")
        w = c * NS + s

        def zb(i, carry):
            zeros_z[i, :] = jnp.zeros((16,), jnp.float32)
            return carry

        lax.fori_loop(0, 256, zb, 0)

        for f, g_hbm in enumerate((g0_hbm, g1_hbm, g2_hbm)):
            _zero_slice(zeros_z, acc, s * RPS)
            plsc.subcore_barrier()

            def win(t, carry, g_hbm=g_hbm):
                rowbase = w * rows_per_worker + t * KW
                pltpu.sync_copy(src_hbm.at[pl.ds(rowbase, KW)], src_w)
                pltpu.sync_copy(dst_hbm.at[pl.ds(rowbase, KW)], dst_w)
                descs = [
                    pltpu.async_copy(g_hbm.at[src_w.at[j]],
                                     rows_v.at[pl.ds(j * 128, 128)], sem)
                    for j in range(KW)
                ]
                for d in descs:
                    d.wait()
                descs = [
                    pltpu.async_copy(rows_v.at[pl.ds(j * 128, 128)],
                                     acc.at[dst_w.at[j]], sem, add=True)
                    for j in range(KW)
                ]
                for d in descs:
                    d.wait()
                return carry

            lax.fori_loop(0, nwin, win, 0)
            plsc.subcore_barrier()
            pltpu.sync_copy(acc.at[pl.ds(s * RPS, RPS)],
                            out_hbm.at[c, s, :, pl.ds(f * FB, FB)])

    k = pl.kernel(
        body,
        out_type=jax.ShapeDtypeStruct((NC, NS, RPS, F), jnp.float32),
        mesh=_mesh(),
        compiler_params=_sc_params,
        scratch_types=[
            pltpu.VMEM((256, FB), jnp.float32),
            pltpu.VMEM((KW, 128), jnp.int32),
            pltpu.VMEM((KW, 128), jnp.int32),
            pltpu.VMEM((WEDGE, FB), jnp.float32),
            pltpu.VMEM_SHARED((NPAD, FB), jnp.float32),
            pltpu.SemaphoreType.DMA,
        ],
    )
    return k(g0, g1, g2, src2d, dst2d)


def _col_spec(rows):
    return pl.BlockSpec((rows, BC), lambda i: (0, i))


_degp_spec = pl.BlockSpec((NC, BC), lambda i: (0, i))
_part_spec = pl.BlockSpec((NC, F, BC), lambda i: (0, 0, i))
_smem_spec = pl.BlockSpec(memory_space=pltpu.SMEM)


def _full_spec(shape):
    nd = len(shape)
    return pl.BlockSpec(shape, lambda i, nd=nd: (0,) * nd)


def _dinv(degp_blk):
    dp = degp_blk
    return lax.rsqrt(1.0 + dp[0:1, :] + dp[1:2, :])


def _prep_body(x_r, pk_r, pkp_r, degp_r, wg_r, bg_r, w1_r,
               comb_o, gate_o, g_o):
    comb = jnp.concatenate([x_r[...], pk_r[...], pkp_r[...]], axis=0)
    comb_o[...] = comb
    gate_o[...] = jax.nn.sigmoid(
        jnp.dot(wg_r[...], comb, precision=PREC) + bg_r[...])
    g_o[...] = jnp.dot(w1_r[...], comb, precision=PREC) * _dinv(degp_r[...])


def _tc_prep(xT, pkT, pkpT, degp, Wg, bgT, W1):
    return pl.pallas_call(
        _prep_body,
        grid=(GRID,),
        in_specs=[_col_spec(32), _col_spec(15), _col_spec(1), _degp_spec,
                  _full_spec((F, F)), _full_spec((F, 1)), _full_spec((F, F))],
        out_specs=(_col_spec(F), _col_spec(F), _col_spec(F)),
        out_shape=(jax.ShapeDtypeStruct((F, NPAD), jnp.float32),
                   jax.ShapeDtypeStruct((F, NPAD), jnp.float32),
                   jax.ShapeDtypeStruct((F, NPAD), jnp.float32)),
    )(xT, pkT, pkpT, degp, Wg, bgT, W1)


def _post_body(p_r, g_r, degp_r, b_r, t_o, st_o):
    i = pl.program_id(0)
    p = p_r[...]
    t = (p[0] + p[1] + g_r[...]) * _dinv(degp_r[...]) + b_r[...]
    t_o[...] = t
    col = lax.broadcasted_iota(jnp.int32, (1, BC), 1)
    tm = jnp.where(col < (N - i * BC), t, 0.0)

    @pl.when(i == 0)
    def _():
        st_o[0, 0] = 0.0
        st_o[0, 1] = 0.0

    st_o[0, 0] += jnp.sum(tm)
    st_o[0, 1] += jnp.sum(tm * tm)


def _tc_post(pT, gT, degp, bT):
    return pl.pallas_call(
        _post_body,
        grid=(GRID,),
        in_specs=[_part_spec, _col_spec(F), _degp_spec, _full_spec((F, 1))],
        out_specs=(_col_spec(F), _smem_spec),
        out_shape=(jax.ShapeDtypeStruct((F, NPAD), jnp.float32),
                   jax.ShapeDtypeStruct((1, 2), jnp.float32)),
    )(pT, gT, degp, bT)


def _apply_body(gated, with_next, t_r, st_r, lw_r, lb_r, aux_r, degp_r,
                *rest):
    inv_cnt = 1.0 / (N * F)
    mean = st_r[0, 0] * inv_cnt
    var = st_r[0, 1] * inv_cnt - mean * mean
    std = jnp.sqrt(jnp.maximum(var, 0.0))
    tn = (t_r[...] - mean) / (std + EPS) * lw_r[...] + lb_r[...]
    r = jnp.maximum(tn, 0.0)
    h = r * aux_r[...] if gated else aux_r[...] + r
    if with_next:
        wn_r, h_o, g_o = rest
        h_o[...] = h
        g_o[...] = jnp.dot(wn_r[...], h, precision=PREC) * _dinv(degp_r[...])
    else:
        (h_o,) = rest
        h_o[...] = h


def _tc_apply(tT, stats, lwT, lbT, auxT, degp, Wn, gated):
    with_next = Wn is not None
    in_specs = [_col_spec(F), _smem_spec, _full_spec((F, 1)),
                _full_spec((F, 1)), _col_spec(F), _degp_spec]
    args = [tT, stats, lwT, lbT, auxT, degp]
    out_specs = [_col_spec(F)]
    out_shape = [jax.ShapeDtypeStruct((F, NPAD), jnp.float32)]
    if with_next:
        in_specs.append(_full_spec((F, F)))
        args.append(Wn)
        out_specs.append(_col_spec(F))
        out_shape.append(jax.ShapeDtypeStruct((F, NPAD), jnp.float32))
    return pl.pallas_call(
        functools.partial(_apply_body, gated, with_next),
        grid=(GRID,),
        in_specs=in_specs,
        out_specs=tuple(out_specs),
        out_shape=tuple(out_shape),
    )(*args)


def _heads_body(h_r, c_r, wp1_r, bp1_r, wp2_r, bp2_r,
                wr1_r, br1_r, wr2_r, br2_r, rw_r, o_r):
    pm = jnp.dot(wp2_r[...], jnp.maximum(
        jnp.dot(wp1_r[...], h_r[...], precision=PREC) + bp1_r[...], 0.0),
        precision=PREC) + bp2_r[0, 0]
    pr = jnp.dot(wr2_r[...], jnp.maximum(
        jnp.dot(wr1_r[...], c_r[...], precision=PREC) + br1_r[...], 0.0),
        precision=PREC) + br2_r[0, 0]
    o_r[...] = pm + rw_r[0, 0] * pr


def _tc_heads(hT, combT, Wp1, bp1T, Wp2, bp2s, Wr1, br1T, Wr2, br2s, rws):
    return pl.pallas_call(
        _heads_body,
        grid=(GRID,),
        in_specs=[_col_spec(F), _col_spec(F),
                  _full_spec((24, F)), _full_spec((24, 1)),
                  _full_spec((1, 24)), _smem_spec,
                  _full_spec((24, F)), _full_spec((24, 1)),
                  _full_spec((1, 24)), _smem_spec, _smem_spec],
        out_specs=_col_spec(1),
        out_shape=jax.ShapeDtypeStruct((1, NPAD), jnp.float32),
    )(hT, combT, Wp1, bp1T, Wp2, bp2s, Wr1, br1T, Wr2, br2s, rws)


def _padT(a):
    return jnp.pad(a.T, ((0, 0), (0, NPAD - a.shape[0])))


def kernel(x, pk_embeddings, pk_predictions, edge_index, Wg, bg, W1, b1,
           W2, b2, W3, b3, ln1_w, ln1_b, ln2_w, ln2_b, ln3_w, ln3_b,
           Wp1, bp1, Wp2, bp2, Wr1, br1, Wr2, br2, rw):
    E = edge_index.shape[1]
    nwin = -(-E // (NW * WEDGE))
    pad = NW * WEDGE * nwin - E
    pad_idx = jnp.arange(pad, dtype=jnp.int32) % 128
    src2d = jnp.concatenate([edge_index[0], pad_idx]).reshape(-1, 128)
    dst2d = jnp.concatenate([edge_index[1], N + pad_idx]).reshape(-1, 128)

    degp = _sc_degree(dst2d, nwin).reshape(NC, NPAD)

    combT, gateT, gT = _tc_prep(
        _padT(x), _padT(pk_embeddings), _padT(pk_predictions), degp,
        Wg, bg.reshape(F, 1), W1)

    layers = [(b1, ln1_w, ln1_b, W2), (b2, ln2_w, ln2_b, W3),
              (b3, ln3_w, ln3_b, None)]
    auxT = gateT
    hT = None
    for li, (bi, lw, lb, Wn) in enumerate(layers):
        g0 = gT[0:16, :].T
        g1 = gT[16:32, :].T
        g2 = gT[32:48, :].T
        part = _sc_scatter(g0, g1, g2, src2d, dst2d, nwin)
        pT = jnp.transpose(part.reshape(NC, NPAD, F), (0, 2, 1))
        tT, stats = _tc_post(pT, gT, degp, bi.reshape(F, 1))
        res = _tc_apply(tT, stats, lw.reshape(F, 1), lb.reshape(F, 1),
                        auxT, degp, Wn, gated=(li == 0))
        if Wn is None:
            hT = res[0]
        else:
            hT, gT = res
        auxT = hT

    oT = _tc_heads(hT, combT, Wp1, bp1.reshape(24, 1), Wp2,
                   bp2.reshape(1, 1), Wr1, br1.reshape(24, 1), Wr2,
                   br2.reshape(1, 1), rw.reshape(1, 1))
    return oT[0, :N].reshape(N, 1)


# speedup vs baseline: 15.2811x; 1.2754x over previous
import functools

import jax
import jax.numpy as jnp
from jax import lax
from jax.experimental import pallas as pl
from jax.experimental.pallas import tpu as pltpu
from jax.experimental.pallas import tpu_sc as plsc

N = 100000
F = 48
FB = 16
NC = 2
NS = 16
NW = NC * NS
NPAD = 102400
RPS = NPAD // NS
KW = 4
WEDGE = KW * 128
BC = 2048
GRID = NPAD // BC
EPS = 1e-5
PREC = lax.Precision.DEFAULT

_mesh = functools.partial(plsc.VectorSubcoreMesh,
                          core_axis_name="c", subcore_axis_name="s")
_sc_params = pltpu.CompilerParams(use_tc_tiling_on_sc=False)


def _zero_slice(zeros_ref, acc_ref, base):
    zn = zeros_ref.shape[0]
    full, rem = RPS // zn, RPS % zn
    for k in range(full):
        pltpu.sync_copy(zeros_ref, acc_ref.at[pl.ds(base + k * zn, zn)])
    if rem:
        pltpu.sync_copy(zeros_ref.at[pl.ds(0, rem)],
                        acc_ref.at[pl.ds(base + full * zn, rem)])


def _sc_degree(dst2d, nwin):
    rows_per_worker = nwin * KW

    def body(dst_hbm, out_hbm, ones_v, zeros_v, dst_w, acc):
        c = lax.axis_index("c")
        s = lax.axis_index("s")
        w = c * NS + s
        for i in range(8):
            ones_v[pl.ds(i * 16, 16)] = jnp.ones((16,), jnp.float32)
        for i in range(32):
            zeros_v[pl.ds(i * 16, 16)] = jnp.zeros((16,), jnp.float32)
        _zero_slice(zeros_v, acc, s * RPS)
        plsc.subcore_barrier()

        def win(t, carry):
            rowbase = w * rows_per_worker + t * KW
            pltpu.sync_copy(dst_hbm.at[pl.ds(rowbase, KW)], dst_w)
            for j in range(KW):
                pltpu.sync_copy(ones_v, acc.at[dst_w.at[j]], add=True)
            return carry

        lax.fori_loop(0, nwin, win, 0)
        plsc.subcore_barrier()
        pltpu.sync_copy(acc.at[pl.ds(s * RPS, RPS)], out_hbm.at[c, s])

    k = pl.kernel(
        body,
        out_type=jax.ShapeDtypeStruct((NC, NS, RPS), jnp.float32),
        mesh=_mesh(),
        compiler_params=_sc_params,
        scratch_types=[
            pltpu.VMEM((128,), jnp.float32),
            pltpu.VMEM((512,), jnp.float32),
            pltpu.VMEM((KW, 128), jnp.int32),
            pltpu.VMEM_SHARED((NPAD,), jnp.float32),
        ],
    )
    return k(dst2d)


def _sc_scatter(g0, g1, g2, src2d, dst2d, nwin):
    rows_per_worker = nwin * KW

    def body(g0_hbm, g1_hbm, g2_hbm, src_hbm, dst_hbm,
             out_hbm, zeros_z, src_w, dst_w, rows_v, acc, gsem, ssem):
        c = lax.axis_index("c")
        s = lax.axis_index("s")
        w = c * NS + s

        def zb(i, carry):
            zeros_z[i, :] = jnp.zeros((16,), jnp.float32)
            return carry

        lax.fori_loop(0, 256, zb, 0)

        wbase = w * rows_per_worker
        for f, g_hbm in enumerate((g0_hbm, g1_hbm, g2_hbm)):
            _zero_slice(zeros_z, acc, s * RPS)
            plsc.subcore_barrier()

            def stage(t, buf):
                pltpu.sync_copy(src_hbm.at[pl.ds(wbase + t * KW, KW)],
                                src_w.at[pl.ds(buf * KW, KW)])
                pltpu.sync_copy(dst_hbm.at[pl.ds(wbase + t * KW, KW)],
                                dst_w.at[pl.ds(buf * KW, KW)])

            def fire_gathers(buf, g_hbm=g_hbm):
                for j in range(KW):
                    pltpu.async_copy(
                        g_hbm.at[src_w.at[buf * KW + j]],
                        rows_v.at[pl.ds((buf * KW + j) * 128, 128)], gsem)

            def wait_gathers(buf, g_hbm=g_hbm):
                for j in range(KW):
                    pltpu.make_async_copy(
                        g_hbm.at[src_w.at[buf * KW + j]],
                        rows_v.at[pl.ds((buf * KW + j) * 128, 128)],
                        gsem).wait()

            stage(0, 0)
            fire_gathers(0)

            def win(t, carry, g_hbm=g_hbm):
                cur = lax.rem(t, 2)
                nxt = 1 - cur
                tn = jnp.minimum(t + 1, nwin - 1)
                stage(tn, nxt)
                wait_gathers(cur)
                fire_gathers(nxt)
                for j in range(KW):
                    pltpu.async_copy(
                        rows_v.at[pl.ds((cur * KW + j) * 128, 128)],
                        acc.at[dst_w.at[cur * KW + j]], ssem, add=True)
                for j in range(KW):
                    pltpu.make_async_copy(
                        rows_v.at[pl.ds((cur * KW + j) * 128, 128)],
                        acc.at[dst_w.at[cur * KW + j]], ssem).wait()
                return carry

            lax.fori_loop(0, nwin, win, 0)
            wait_gathers(nwin % 2)
            plsc.subcore_barrier()
            pltpu.sync_copy(acc.at[pl.ds(s * RPS, RPS)],
                            out_hbm.at[c, s, :, pl.ds(f * FB, FB)])

    k = pl.kernel(
        body,
        out_type=jax.ShapeDtypeStruct((NC, NS, RPS, F), jnp.float32),
        mesh=_mesh(),
        compiler_params=_sc_params,
        scratch_types=[
            pltpu.VMEM((256, FB), jnp.float32),
            pltpu.VMEM((2 * KW, 128), jnp.int32),
            pltpu.VMEM((2 * KW, 128), jnp.int32),
            pltpu.VMEM((2 * WEDGE, FB), jnp.float32),
            pltpu.VMEM_SHARED((NPAD, FB), jnp.float32),
            pltpu.SemaphoreType.DMA,
            pltpu.SemaphoreType.DMA,
        ],
    )
    return k(g0, g1, g2, src2d, dst2d)


def _col_spec(rows):
    return pl.BlockSpec((rows, BC), lambda i: (0, i))


_degp_spec = pl.BlockSpec((NC, BC), lambda i: (0, i))
_part_spec = pl.BlockSpec((NC, F, BC), lambda i: (0, 0, i))
_smem_spec = pl.BlockSpec(memory_space=pltpu.SMEM)


def _full_spec(shape):
    nd = len(shape)
    return pl.BlockSpec(shape, lambda i, nd=nd: (0,) * nd)


def _dinv(degp_blk):
    dp = degp_blk
    return lax.rsqrt(1.0 + dp[0:1, :] + dp[1:2, :])


def _prep_body(x_r, pk_r, pkp_r, degp_r, wg_r, bg_r, w1_r,
               comb_o, gate_o, g_o):
    comb = jnp.concatenate([x_r[...], pk_r[...], pkp_r[...]], axis=0)
    comb_o[...] = comb
    gate_o[...] = jax.nn.sigmoid(
        jnp.dot(wg_r[...], comb, precision=PREC) + bg_r[...])
    g_o[...] = jnp.dot(w1_r[...], comb, precision=PREC) * _dinv(degp_r[...])


def _tc_prep(xT, pkT, pkpT, degp, Wg, bgT, W1):
    return pl.pallas_call(
        _prep_body,
        grid=(GRID,),
        in_specs=[_col_spec(32), _col_spec(15), _col_spec(1), _degp_spec,
                  _full_spec((F, F)), _full_spec((F, 1)), _full_spec((F, F))],
        out_specs=(_col_spec(F), _col_spec(F), _col_spec(F)),
        out_shape=(jax.ShapeDtypeStruct((F, NPAD), jnp.float32),
                   jax.ShapeDtypeStruct((F, NPAD), jnp.float32),
                   jax.ShapeDtypeStruct((F, NPAD), jnp.float32)),
    )(xT, pkT, pkpT, degp, Wg, bgT, W1)


def _post_body(p_r, g_r, degp_r, b_r, t_o, st_o):
    i = pl.program_id(0)
    p = p_r[...]
    t = (p[0] + p[1] + g_r[...]) * _dinv(degp_r[...]) + b_r[...]
    t_o[...] = t
    col = lax.broadcasted_iota(jnp.int32, (1, BC), 1)
    tm = jnp.where(col < (N - i * BC), t, 0.0)

    @pl.when(i == 0)
    def _():
        st_o[0, 0] = 0.0
        st_o[0, 1] = 0.0

    st_o[0, 0] += jnp.sum(tm)
    st_o[0, 1] += jnp.sum(tm * tm)


def _tc_post(pT, gT, degp, bT):
    return pl.pallas_call(
        _post_body,
        grid=(GRID,),
        in_specs=[_part_spec, _col_spec(F), _degp_spec, _full_spec((F, 1))],
        out_specs=(_col_spec(F), _smem_spec),
        out_shape=(jax.ShapeDtypeStruct((F, NPAD), jnp.float32),
                   jax.ShapeDtypeStruct((1, 2), jnp.float32)),
    )(pT, gT, degp, bT)


def _apply_body(gated, with_next, t_r, st_r, lw_r, lb_r, aux_r, degp_r,
                *rest):
    inv_cnt = 1.0 / (N * F)
    mean = st_r[0, 0] * inv_cnt
    var = st_r[0, 1] * inv_cnt - mean * mean
    std = jnp.sqrt(jnp.maximum(var, 0.0))
    tn = (t_r[...] - mean) / (std + EPS) * lw_r[...] + lb_r[...]
    r = jnp.maximum(tn, 0.0)
    h = r * aux_r[...] if gated else aux_r[...] + r
    if with_next:
        wn_r, h_o, g_o = rest
        h_o[...] = h
        g_o[...] = jnp.dot(wn_r[...], h, precision=PREC) * _dinv(degp_r[...])
    else:
        (h_o,) = rest
        h_o[...] = h


def _tc_apply(tT, stats, lwT, lbT, auxT, degp, Wn, gated):
    with_next = Wn is not None
    in_specs = [_col_spec(F), _smem_spec, _full_spec((F, 1)),
                _full_spec((F, 1)), _col_spec(F), _degp_spec]
    args = [tT, stats, lwT, lbT, auxT, degp]
    out_specs = [_col_spec(F)]
    out_shape = [jax.ShapeDtypeStruct((F, NPAD), jnp.float32)]
    if with_next:
        in_specs.append(_full_spec((F, F)))
        args.append(Wn)
        out_specs.append(_col_spec(F))
        out_shape.append(jax.ShapeDtypeStruct((F, NPAD), jnp.float32))
    return pl.pallas_call(
        functools.partial(_apply_body, gated, with_next),
        grid=(GRID,),
        in_specs=in_specs,
        out_specs=tuple(out_specs),
        out_shape=tuple(out_shape),
    )(*args)


def _heads_body(h_r, c_r, wp1_r, bp1_r, wp2_r, bp2_r,
                wr1_r, br1_r, wr2_r, br2_r, rw_r, o_r):
    pm = jnp.dot(wp2_r[...], jnp.maximum(
        jnp.dot(wp1_r[...], h_r[...], precision=PREC) + bp1_r[...], 0.0),
        precision=PREC) + bp2_r[0, 0]
    pr = jnp.dot(wr2_r[...], jnp.maximum(
        jnp.dot(wr1_r[...], c_r[...], precision=PREC) + br1_r[...], 0.0),
        precision=PREC) + br2_r[0, 0]
    o_r[...] = pm + rw_r[0, 0] * pr


def _tc_heads(hT, combT, Wp1, bp1T, Wp2, bp2s, Wr1, br1T, Wr2, br2s, rws):
    return pl.pallas_call(
        _heads_body,
        grid=(GRID,),
        in_specs=[_col_spec(F), _col_spec(F),
                  _full_spec((24, F)), _full_spec((24, 1)),
                  _full_spec((1, 24)), _smem_spec,
                  _full_spec((24, F)), _full_spec((24, 1)),
                  _full_spec((1, 24)), _smem_spec, _smem_spec],
        out_specs=_col_spec(1),
        out_shape=jax.ShapeDtypeStruct((1, NPAD), jnp.float32),
    )(hT, combT, Wp1, bp1T, Wp2, bp2s, Wr1, br1T, Wr2, br2s, rws)


def _padT(a):
    return jnp.pad(a.T, ((0, 0), (0, NPAD - a.shape[0])))


def kernel(x, pk_embeddings, pk_predictions, edge_index, Wg, bg, W1, b1,
           W2, b2, W3, b3, ln1_w, ln1_b, ln2_w, ln2_b, ln3_w, ln3_b,
           Wp1, bp1, Wp2, bp2, Wr1, br1, Wr2, br2, rw):
    E = edge_index.shape[1]
    nwin = -(-E // (NW * WEDGE))
    pad = NW * WEDGE * nwin - E
    pad_idx = jnp.arange(pad, dtype=jnp.int32) % 128
    src2d = jnp.concatenate([edge_index[0], pad_idx]).reshape(-1, 128)
    dst2d = jnp.concatenate([edge_index[1], N + pad_idx]).reshape(-1, 128)

    degp = _sc_degree(dst2d, nwin).reshape(NC, NPAD)

    combT, gateT, gT = _tc_prep(
        _padT(x), _padT(pk_embeddings), _padT(pk_predictions), degp,
        Wg, bg.reshape(F, 1), W1)

    layers = [(b1, ln1_w, ln1_b, W2), (b2, ln2_w, ln2_b, W3),
              (b3, ln3_w, ln3_b, None)]
    auxT = gateT
    hT = None
    for li, (bi, lw, lb, Wn) in enumerate(layers):
        g0 = gT[0:16, :].T
        g1 = gT[16:32, :].T
        g2 = gT[32:48, :].T
        part = _sc_scatter(g0, g1, g2, src2d, dst2d, nwin)
        pT = jnp.transpose(part, (0, 3, 1, 2)).reshape(NC, F, NPAD)
        tT, stats = _tc_post(pT, gT, degp, bi.reshape(F, 1))
        res = _tc_apply(tT, stats, lw.reshape(F, 1), lb.reshape(F, 1),
                        auxT, degp, Wn, gated=(li == 0))
        if Wn is None:
            hT = res[0]
        else:
            hT, gT = res
        auxT = hT

    oT = _tc_heads(hT, combT, Wp1, bp1.reshape(24, 1), Wp2,
                   bp2.reshape(1, 1), Wr1, br1.reshape(24, 1), Wr2,
                   br2.reshape(1, 1), rw.reshape(1, 1))
    return oT[0, :N].reshape(N, 1)


# speedup vs baseline: 16.6454x; 1.0893x over previous
import functools

import jax
import jax.numpy as jnp
from jax import lax
from jax.experimental import pallas as pl
from jax.experimental.pallas import tpu as pltpu
from jax.experimental.pallas import tpu_sc as plsc

N = 100000
F = 48
FB = 16
NC = 2
NS = 16
NW = NC * NS
NPAD = 100352
RPS = NPAD // NS
KW = 5
WEDGE = KW * 128
IB = 2 * KW
BC = 2048
GRID = NPAD // BC
EPS = 1e-5
PREC = lax.Precision.DEFAULT

_mesh = functools.partial(plsc.VectorSubcoreMesh,
                          core_axis_name="c", subcore_axis_name="s")
_sc_params = pltpu.CompilerParams(use_tc_tiling_on_sc=False)


def _zero_slice(zeros_ref, acc_ref, base):
    zn = zeros_ref.shape[0]
    full, rem = RPS // zn, RPS % zn
    for k in range(full):
        pltpu.sync_copy(zeros_ref, acc_ref.at[pl.ds(base + k * zn, zn)])
    if rem:
        pltpu.sync_copy(zeros_ref.at[pl.ds(0, rem)],
                        acc_ref.at[pl.ds(base + full * zn, rem)])


def _sc_degree(comb2d, nwin):
    def body(comb_hbm, out_hbm, ones_v, zeros_v, dst_w, acc):
        c = lax.axis_index("c")
        s = lax.axis_index("s")
        w = c * NS + s
        for i in range(8):
            ones_v[pl.ds(i * 16, 16)] = jnp.ones((16,), jnp.float32)
        for i in range(32):
            zeros_v[pl.ds(i * 16, 16)] = jnp.zeros((16,), jnp.float32)
        _zero_slice(zeros_v, acc, s * RPS)
        plsc.subcore_barrier()

        def win(t, carry):
            rowbase = (w * nwin + t) * IB + KW
            pltpu.sync_copy(comb_hbm.at[pl.ds(rowbase, KW)], dst_w)
            for j in range(KW):
                pltpu.sync_copy(ones_v, acc.at[dst_w.at[j]], add=True)
            return carry

        lax.fori_loop(0, nwin, win, 0)
        plsc.subcore_barrier()
        pltpu.sync_copy(acc.at[pl.ds(s * RPS, RPS)], out_hbm.at[c, s])

    k = pl.kernel(
        body,
        out_type=jax.ShapeDtypeStruct((NC, NS, RPS), jnp.float32),
        mesh=_mesh(),
        compiler_params=_sc_params,
        scratch_types=[
            pltpu.VMEM((128,), jnp.float32),
            pltpu.VMEM((512,), jnp.float32),
            pltpu.VMEM((KW, 128), jnp.int32),
            pltpu.VMEM_SHARED((NPAD,), jnp.float32),
        ],
    )
    return k(comb2d)


def _sc_scatter(g0, g1, g2, comb2d, nwin):
    def body(g0_hbm, g1_hbm, g2_hbm, comb_hbm,
             out_hbm, zeros_z, idx_w, rows_v, acc, gsem, ssem):
        c = lax.axis_index("c")
        s = lax.axis_index("s")
        w = c * NS + s

        def zb(i, carry):
            zeros_z[i, :] = jnp.zeros((16,), jnp.float32)
            return carry

        lax.fori_loop(0, 256, zb, 0)

        for f, g_hbm in enumerate((g0_hbm, g1_hbm, g2_hbm)):
            _zero_slice(zeros_z, acc, s * RPS)
            plsc.subcore_barrier()

            def stage(t, buf):
                pltpu.sync_copy(comb_hbm.at[pl.ds((w * nwin + t) * IB, IB)],
                                idx_w.at[pl.ds(buf * IB, IB)])

            def fire_gathers(buf, g_hbm=g_hbm):
                for j in range(KW):
                    pltpu.async_copy(
                        g_hbm.at[idx_w.at[buf * IB + j]],
                        rows_v.at[pl.ds((buf * KW + j) * 128, 128)], gsem)

            def wait_gathers(buf, g_hbm=g_hbm):
                for j in range(KW):
                    pltpu.make_async_copy(
                        g_hbm.at[idx_w.at[buf * IB + j]],
                        rows_v.at[pl.ds((buf * KW + j) * 128, 128)],
                        gsem).wait()

            def fire_scatters(buf):
                for j in range(KW):
                    pltpu.async_copy(
                        rows_v.at[pl.ds((buf * KW + j) * 128, 128)],
                        acc.at[idx_w.at[buf * IB + KW + j]], ssem, add=True)

            def wait_scatters(buf):
                for j in range(KW):
                    pltpu.make_async_copy(
                        rows_v.at[pl.ds((buf * KW + j) * 128, 128)],
                        acc.at[idx_w.at[buf * IB + KW + j]], ssem).wait()

            stage(0, 0)
            fire_gathers(0)

            def win(t, carry, g_hbm=g_hbm):
                cur = lax.rem(t, 2)
                nxt = 1 - cur

                @pl.when(t > 0)
                def _():
                    wait_scatters(nxt)

                stage(jnp.minimum(t + 1, nwin - 1), nxt)
                wait_gathers(cur)
                fire_gathers(nxt)
                fire_scatters(cur)
                return carry

            lax.fori_loop(0, nwin, win, 0)
            last = (nwin - 1) % 2
            wait_scatters(last)
            wait_gathers(1 - last)
            plsc.subcore_barrier()
            pltpu.sync_copy(acc.at[pl.ds(s * RPS, RPS)],
                            out_hbm.at[c, s, :, pl.ds(f * FB, FB)])

    k = pl.kernel(
        body,
        out_type=jax.ShapeDtypeStruct((NC, NS, RPS, F), jnp.float32),
        mesh=_mesh(),
        compiler_params=_sc_params,
        scratch_types=[
            pltpu.VMEM((256, FB), jnp.float32),
            pltpu.VMEM((2 * IB, 128), jnp.int32),
            pltpu.VMEM((2 * WEDGE, FB), jnp.float32),
            pltpu.VMEM_SHARED((NPAD, FB), jnp.float32),
            pltpu.SemaphoreType.DMA,
            pltpu.SemaphoreType.DMA,
        ],
    )
    return k(g0, g1, g2, comb2d)


def _col_spec(rows):
    return pl.BlockSpec((rows, BC), lambda i: (0, i))


_degp_spec = pl.BlockSpec((NC, BC), lambda i: (0, i))
_part_spec = pl.BlockSpec((NC, F, BC), lambda i: (0, 0, i))
_smem_spec = pl.BlockSpec(memory_space=pltpu.SMEM)


def _full_spec(shape):
    nd = len(shape)
    return pl.BlockSpec(shape, lambda i, nd=nd: (0,) * nd)


def _dinv(degp_blk):
    dp = degp_blk
    return lax.rsqrt(1.0 + dp[0:1, :] + dp[1:2, :])


def _prep_body(x_r, pk_r, pkp_r, degp_r, wg_r, bg_r, w1_r,
               comb_o, gate_o, g_o):
    comb = jnp.concatenate([x_r[...], pk_r[...], pkp_r[...]], axis=0)
    comb_o[...] = comb
    gate_o[...] = jax.nn.sigmoid(
        jnp.dot(wg_r[...], comb, precision=PREC) + bg_r[...])
    g_o[...] = jnp.dot(w1_r[...], comb, precision=PREC) * _dinv(degp_r[...])


def _tc_prep(xT, pkT, pkpT, degp, Wg, bgT, W1):
    return pl.pallas_call(
        _prep_body,
        grid=(GRID,),
        in_specs=[_col_spec(32), _col_spec(15), _col_spec(1), _degp_spec,
                  _full_spec((F, F)), _full_spec((F, 1)), _full_spec((F, F))],
        out_specs=(_col_spec(F), _col_spec(F), _col_spec(F)),
        out_shape=(jax.ShapeDtypeStruct((F, NPAD), jnp.float32),
                   jax.ShapeDtypeStruct((F, NPAD), jnp.float32),
                   jax.ShapeDtypeStruct((F, NPAD), jnp.float32)),
    )(xT, pkT, pkpT, degp, Wg, bgT, W1)


def _post_body(p_r, g_r, degp_r, b_r, t_o, st_o):
    i = pl.program_id(0)
    p = p_r[...]
    t = (p[0] + p[1] + g_r[...]) * _dinv(degp_r[...]) + b_r[...]
    t_o[...] = t
    col = lax.broadcasted_iota(jnp.int32, (1, BC), 1)
    tm = jnp.where(col < (N - i * BC), t, 0.0)

    @pl.when(i == 0)
    def _():
        st_o[0, 0] = 0.0
        st_o[0, 1] = 0.0

    st_o[0, 0] += jnp.sum(tm)
    st_o[0, 1] += jnp.sum(tm * tm)


def _tc_post(pT, gT, degp, bT):
    return pl.pallas_call(
        _post_body,
        grid=(GRID,),
        in_specs=[_part_spec, _col_spec(F), _degp_spec, _full_spec((F, 1))],
        out_specs=(_col_spec(F), _smem_spec),
        out_shape=(jax.ShapeDtypeStruct((F, NPAD), jnp.float32),
                   jax.ShapeDtypeStruct((1, 2), jnp.float32)),
    )(pT, gT, degp, bT)


def _apply_body(gated, with_next, t_r, st_r, lw_r, lb_r, aux_r, degp_r,
                *rest):
    inv_cnt = 1.0 / (N * F)
    mean = st_r[0, 0] * inv_cnt
    var = st_r[0, 1] * inv_cnt - mean * mean
    std = jnp.sqrt(jnp.maximum(var, 0.0))
    tn = (t_r[...] - mean) / (std + EPS) * lw_r[...] + lb_r[...]
    r = jnp.maximum(tn, 0.0)
    h = r * aux_r[...] if gated else aux_r[...] + r
    if with_next:
        wn_r, h_o, g_o = rest
        h_o[...] = h
        g_o[...] = jnp.dot(wn_r[...], h, precision=PREC) * _dinv(degp_r[...])
    else:
        (h_o,) = rest
        h_o[...] = h


def _tc_apply(tT, stats, lwT, lbT, auxT, degp, Wn, gated):
    with_next = Wn is not None
    in_specs = [_col_spec(F), _smem_spec, _full_spec((F, 1)),
                _full_spec((F, 1)), _col_spec(F), _degp_spec]
    args = [tT, stats, lwT, lbT, auxT, degp]
    out_specs = [_col_spec(F)]
    out_shape = [jax.ShapeDtypeStruct((F, NPAD), jnp.float32)]
    if with_next:
        in_specs.append(_full_spec((F, F)))
        args.append(Wn)
        out_specs.append(_col_spec(F))
        out_shape.append(jax.ShapeDtypeStruct((F, NPAD), jnp.float32))
    return pl.pallas_call(
        functools.partial(_apply_body, gated, with_next),
        grid=(GRID,),
        in_specs=in_specs,
        out_specs=tuple(out_specs),
        out_shape=tuple(out_shape),
    )(*args)


def _heads_body(h_r, c_r, wp1_r, bp1_r, wp2_r, bp2_r,
                wr1_r, br1_r, wr2_r, br2_r, rw_r, o_r):
    pm = jnp.dot(wp2_r[...], jnp.maximum(
        jnp.dot(wp1_r[...], h_r[...], precision=PREC) + bp1_r[...], 0.0),
        precision=PREC) + bp2_r[0, 0]
    pr = jnp.dot(wr2_r[...], jnp.maximum(
        jnp.dot(wr1_r[...], c_r[...], precision=PREC) + br1_r[...], 0.0),
        precision=PREC) + br2_r[0, 0]
    o_r[...] = pm + rw_r[0, 0] * pr


def _tc_heads(hT, combT, Wp1, bp1T, Wp2, bp2s, Wr1, br1T, Wr2, br2s, rws):
    return pl.pallas_call(
        _heads_body,
        grid=(GRID,),
        in_specs=[_col_spec(F), _col_spec(F),
                  _full_spec((24, F)), _full_spec((24, 1)),
                  _full_spec((1, 24)), _smem_spec,
                  _full_spec((24, F)), _full_spec((24, 1)),
                  _full_spec((1, 24)), _smem_spec, _smem_spec],
        out_specs=_col_spec(1),
        out_shape=jax.ShapeDtypeStruct((1, NPAD), jnp.float32),
    )(hT, combT, Wp1, bp1T, Wp2, bp2s, Wr1, br1T, Wr2, br2s, rws)


def _padT(a):
    return jnp.pad(a.T, ((0, 0), (0, NPAD - a.shape[0])))


def kernel(x, pk_embeddings, pk_predictions, edge_index, Wg, bg, W1, b1,
           W2, b2, W3, b3, ln1_w, ln1_b, ln2_w, ln2_b, ln3_w, ln3_b,
           Wp1, bp1, Wp2, bp2, Wr1, br1, Wr2, br2, rw):
    E = edge_index.shape[1]
    nwin = -(-E // (NW * WEDGE))
    pad = NW * WEDGE * nwin - E
    pad_idx = jnp.arange(pad, dtype=jnp.int32) % 128
    src3 = jnp.concatenate([edge_index[0], pad_idx]).reshape(-1, KW, 128)
    dst3 = jnp.concatenate([edge_index[1], N + pad_idx]).reshape(-1, KW, 128)
    comb2d = jnp.concatenate([src3, dst3], axis=1).reshape(-1, 128)

    degp = _sc_degree(comb2d, nwin).reshape(NC, NPAD)

    combT, gateT, gT = _tc_prep(
        _padT(x), _padT(pk_embeddings), _padT(pk_predictions), degp,
        Wg, bg.reshape(F, 1), W1)

    layers = [(b1, ln1_w, ln1_b, W2), (b2, ln2_w, ln2_b, W3),
              (b3, ln3_w, ln3_b, None)]
    auxT = gateT
    hT = None
    for li, (bi, lw, lb, Wn) in enumerate(layers):
        g0 = gT[0:16, :].T
        g1 = gT[16:32, :].T
        g2 = gT[32:48, :].T
        part = _sc_scatter(g0, g1, g2, comb2d, nwin)
        pT = jnp.transpose(part, (0, 3, 1, 2)).reshape(NC, F, NPAD)
        tT, stats = _tc_post(pT, gT, degp, bi.reshape(F, 1))
        res = _tc_apply(tT, stats, lw.reshape(F, 1), lb.reshape(F, 1),
                        auxT, degp, Wn, gated=(li == 0))
        if Wn is None:
            hT = res[0]
        else:
            hT, gT = res
        auxT = hT

    oT = _tc_heads(hT, combT, Wp1, bp1.reshape(24, 1), Wp2,
                   bp2.reshape(1, 1), Wr1, br1.reshape(24, 1), Wr2,
                   br2.reshape(1, 1), rw.reshape(1, 1))
    return oT[0, :N].reshape(N, 1)


# speedup vs baseline: 18.3157x; 1.1003x over previous
import functools

import jax
import jax.numpy as jnp
from jax import lax
from jax.experimental import pallas as pl
from jax.experimental.pallas import tpu as pltpu
from jax.experimental.pallas import tpu_sc as plsc

N = 100000
F = 48
FB = 16
NC = 2
NS = 16
NW = NC * NS
NPAD = 100352
RPS = NPAD // NS
KW = 5
WEDGE = KW * 128
IB = 2 * KW
BC = 2048
GRID = NPAD // BC
EPS = 1e-5
PREC = lax.Precision.DEFAULT

_mesh = functools.partial(plsc.VectorSubcoreMesh,
                          core_axis_name="c", subcore_axis_name="s")
_sc_params = pltpu.CompilerParams(use_tc_tiling_on_sc=False)


def _zero_slice(zeros_ref, acc_ref, base):
    zn = zeros_ref.shape[0]
    full, rem = RPS // zn, RPS % zn
    for k in range(full):
        pltpu.sync_copy(zeros_ref, acc_ref.at[pl.ds(base + k * zn, zn)])
    if rem:
        pltpu.sync_copy(zeros_ref.at[pl.ds(0, rem)],
                        acc_ref.at[pl.ds(base + full * zn, rem)])


def _sc_degree(comb2d, nwin):
    def body(comb_hbm, out_hbm, ones_v, zeros_v, dst_w, acc):
        c = lax.axis_index("c")
        s = lax.axis_index("s")
        w = c * NS + s
        for i in range(8):
            ones_v[pl.ds(i * 16, 16)] = jnp.ones((16,), jnp.float32)
        for i in range(32):
            zeros_v[pl.ds(i * 16, 16)] = jnp.zeros((16,), jnp.float32)
        _zero_slice(zeros_v, acc, s * RPS)
        plsc.subcore_barrier()

        def win(t, carry):
            rowbase = (w * nwin + t) * IB + KW
            pltpu.sync_copy(comb_hbm.at[pl.ds(rowbase, KW)], dst_w)
            for j in range(KW):
                pltpu.sync_copy(ones_v, acc.at[dst_w.at[j]], add=True)
            return carry

        lax.fori_loop(0, nwin, win, 0)
        plsc.subcore_barrier()
        pltpu.sync_copy(acc.at[pl.ds(s * RPS, RPS)], out_hbm.at[c, s])

    k = pl.kernel(
        body,
        out_type=jax.ShapeDtypeStruct((NC, NS, RPS), jnp.float32),
        mesh=_mesh(),
        compiler_params=_sc_params,
        scratch_types=[
            pltpu.VMEM((128,), jnp.float32),
            pltpu.VMEM((512,), jnp.float32),
            pltpu.VMEM((KW, 128), jnp.int32),
            pltpu.VMEM_SHARED((NPAD,), jnp.float32),
        ],
    )
    return k(comb2d)


def _sc_scatter(g3, comb2d, nwin):
    def body(g3_hbm, comb_hbm,
             out_hbm, zeros_z, idx_w, rows_v, acc, gsem, ssem):
        c = lax.axis_index("c")
        s = lax.axis_index("s")
        w = c * NS + s

        def zb(i, carry):
            zeros_z[i, :] = jnp.zeros((16,), jnp.float32)
            return carry

        lax.fori_loop(0, 256, zb, 0)

        for f in range(3):
            g_hbm = g3_hbm.at[f]
            _zero_slice(zeros_z, acc, s * RPS)
            plsc.subcore_barrier()

            def stage(t, buf):
                pltpu.sync_copy(comb_hbm.at[pl.ds((w * nwin + t) * IB, IB)],
                                idx_w.at[pl.ds(buf * IB, IB)])

            def fire_gathers(buf, g_hbm=g_hbm):
                for j in range(KW):
                    pltpu.async_copy(
                        g_hbm.at[idx_w.at[buf * IB + j]],
                        rows_v.at[pl.ds((buf * KW + j) * 128, 128)], gsem)

            def wait_gathers(buf, g_hbm=g_hbm):
                for j in range(KW):
                    pltpu.make_async_copy(
                        g_hbm.at[idx_w.at[buf * IB + j]],
                        rows_v.at[pl.ds((buf * KW + j) * 128, 128)],
                        gsem).wait()

            def fire_scatters(buf):
                for j in range(KW):
                    pltpu.async_copy(
                        rows_v.at[pl.ds((buf * KW + j) * 128, 128)],
                        acc.at[idx_w.at[buf * IB + KW + j]], ssem, add=True)

            def wait_scatters(buf):
                for j in range(KW):
                    pltpu.make_async_copy(
                        rows_v.at[pl.ds((buf * KW + j) * 128, 128)],
                        acc.at[idx_w.at[buf * IB + KW + j]], ssem).wait()

            stage(0, 0)
            fire_gathers(0)

            def win(t, carry, g_hbm=g_hbm):
                cur = lax.rem(t, 2)
                nxt = 1 - cur

                @pl.when(t > 0)
                def _():
                    wait_scatters(nxt)

                stage(jnp.minimum(t + 1, nwin - 1), nxt)
                wait_gathers(cur)
                fire_gathers(nxt)
                fire_scatters(cur)
                return carry

            lax.fori_loop(0, nwin, win, 0)
            last = (nwin - 1) % 2
            wait_scatters(last)
            wait_gathers(1 - last)
            plsc.subcore_barrier()
            pltpu.sync_copy(
                acc.at[pl.ds(s * RPS, RPS)],
                out_hbm.at[c, pl.ds(s * RPS, RPS), pl.ds(f * FB, FB)])

    k = pl.kernel(
        body,
        out_type=jax.ShapeDtypeStruct((NC, NPAD, F), jnp.float32),
        mesh=_mesh(),
        compiler_params=_sc_params,
        scratch_types=[
            pltpu.VMEM((256, FB), jnp.float32),
            pltpu.VMEM((2 * IB, 128), jnp.int32),
            pltpu.VMEM((2 * WEDGE, FB), jnp.float32),
            pltpu.VMEM_SHARED((NPAD, FB), jnp.float32),
            pltpu.SemaphoreType.DMA,
            pltpu.SemaphoreType.DMA,
        ],
    )
    return k(g3, comb2d)


def _col_spec(rows):
    return pl.BlockSpec((rows, BC), lambda i: (0, i))


_degp_spec = pl.BlockSpec((NC, BC), lambda i: (0, i))
_part_spec = pl.BlockSpec((NC, F, BC), lambda i: (0, 0, i))
_smem_spec = pl.BlockSpec(memory_space=pltpu.SMEM)


def _full_spec(shape):
    nd = len(shape)
    return pl.BlockSpec(shape, lambda i, nd=nd: (0,) * nd)


def _dinv(degp_blk):
    dp = degp_blk
    return lax.rsqrt(1.0 + dp[0:1, :] + dp[1:2, :])


def _prep_body(x_r, pk_r, pkp_r, degp_r, wg_r, bg_r, w1_r,
               comb_o, gate_o, g_o):
    comb = jnp.concatenate([x_r[...], pk_r[...], pkp_r[...]], axis=0)
    comb_o[...] = comb
    gate_o[...] = jax.nn.sigmoid(
        jnp.dot(wg_r[...], comb, precision=PREC) + bg_r[...])
    g_o[...] = jnp.dot(w1_r[...], comb, precision=PREC) * _dinv(degp_r[...])


def _tc_prep(xT, pkT, pkpT, degp, Wg, bgT, W1):
    return pl.pallas_call(
        _prep_body,
        grid=(GRID,),
        in_specs=[_col_spec(32), _col_spec(15), _col_spec(1), _degp_spec,
                  _full_spec((F, F)), _full_spec((F, 1)), _full_spec((F, F))],
        out_specs=(_col_spec(F), _col_spec(F), _col_spec(F)),
        out_shape=(jax.ShapeDtypeStruct((F, NPAD), jnp.float32),
                   jax.ShapeDtypeStruct((F, NPAD), jnp.float32),
                   jax.ShapeDtypeStruct((F, NPAD), jnp.float32)),
    )(xT, pkT, pkpT, degp, Wg, bgT, W1)


def _post_body(p_r, g_r, degp_r, b_r, t_o, st_o):
    i = pl.program_id(0)
    p = p_r[...]
    t = (p[0] + p[1] + g_r[...]) * _dinv(degp_r[...]) + b_r[...]
    t_o[...] = t
    col = lax.broadcasted_iota(jnp.int32, (1, BC), 1)
    tm = jnp.where(col < (N - i * BC), t, 0.0)

    @pl.when(i == 0)
    def _():
        st_o[0, 0] = 0.0
        st_o[0, 1] = 0.0

    st_o[0, 0] += jnp.sum(tm)
    st_o[0, 1] += jnp.sum(tm * tm)


def _tc_post(pT, gT, degp, bT):
    return pl.pallas_call(
        _post_body,
        grid=(GRID,),
        in_specs=[_part_spec, _col_spec(F), _degp_spec, _full_spec((F, 1))],
        out_specs=(_col_spec(F), _smem_spec),
        out_shape=(jax.ShapeDtypeStruct((F, NPAD), jnp.float32),
                   jax.ShapeDtypeStruct((1, 2), jnp.float32)),
    )(pT, gT, degp, bT)


def _apply_body(gated, with_next, t_r, st_r, lw_r, lb_r, aux_r, degp_r,
                *rest):
    inv_cnt = 1.0 / (N * F)
    mean = st_r[0, 0] * inv_cnt
    var = st_r[0, 1] * inv_cnt - mean * mean
    std = jnp.sqrt(jnp.maximum(var, 0.0))
    tn = (t_r[...] - mean) / (std + EPS) * lw_r[...] + lb_r[...]
    r = jnp.maximum(tn, 0.0)
    h = r * aux_r[...] if gated else aux_r[...] + r
    if with_next:
        wn_r, h_o, g_o = rest
        h_o[...] = h
        g_o[...] = jnp.dot(wn_r[...], h, precision=PREC) * _dinv(degp_r[...])
    else:
        (h_o,) = rest
        h_o[...] = h


def _tc_apply(tT, stats, lwT, lbT, auxT, degp, Wn, gated):
    with_next = Wn is not None
    in_specs = [_col_spec(F), _smem_spec, _full_spec((F, 1)),
                _full_spec((F, 1)), _col_spec(F), _degp_spec]
    args = [tT, stats, lwT, lbT, auxT, degp]
    out_specs = [_col_spec(F)]
    out_shape = [jax.ShapeDtypeStruct((F, NPAD), jnp.float32)]
    if with_next:
        in_specs.append(_full_spec((F, F)))
        args.append(Wn)
        out_specs.append(_col_spec(F))
        out_shape.append(jax.ShapeDtypeStruct((F, NPAD), jnp.float32))
    return pl.pallas_call(
        functools.partial(_apply_body, gated, with_next),
        grid=(GRID,),
        in_specs=in_specs,
        out_specs=tuple(out_specs),
        out_shape=tuple(out_shape),
    )(*args)


def _heads_body(h_r, c_r, wp1_r, bp1_r, wp2_r, bp2_r,
                wr1_r, br1_r, wr2_r, br2_r, rw_r, o_r):
    pm = jnp.dot(wp2_r[...], jnp.maximum(
        jnp.dot(wp1_r[...], h_r[...], precision=PREC) + bp1_r[...], 0.0),
        precision=PREC) + bp2_r[0, 0]
    pr = jnp.dot(wr2_r[...], jnp.maximum(
        jnp.dot(wr1_r[...], c_r[...], precision=PREC) + br1_r[...], 0.0),
        precision=PREC) + br2_r[0, 0]
    o_r[...] = pm + rw_r[0, 0] * pr


def _tc_heads(hT, combT, Wp1, bp1T, Wp2, bp2s, Wr1, br1T, Wr2, br2s, rws):
    return pl.pallas_call(
        _heads_body,
        grid=(GRID,),
        in_specs=[_col_spec(F), _col_spec(F),
                  _full_spec((24, F)), _full_spec((24, 1)),
                  _full_spec((1, 24)), _smem_spec,
                  _full_spec((24, F)), _full_spec((24, 1)),
                  _full_spec((1, 24)), _smem_spec, _smem_spec],
        out_specs=_col_spec(1),
        out_shape=jax.ShapeDtypeStruct((1, NPAD), jnp.float32),
    )(hT, combT, Wp1, bp1T, Wp2, bp2s, Wr1, br1T, Wr2, br2s, rws)


def _padT(a):
    return jnp.pad(a.T, ((0, 0), (0, NPAD - a.shape[0])))


def kernel(x, pk_embeddings, pk_predictions, edge_index, Wg, bg, W1, b1,
           W2, b2, W3, b3, ln1_w, ln1_b, ln2_w, ln2_b, ln3_w, ln3_b,
           Wp1, bp1, Wp2, bp2, Wr1, br1, Wr2, br2, rw):
    E = edge_index.shape[1]
    nwin = -(-E // (NW * WEDGE))
    pad = NW * WEDGE * nwin - E
    pad_idx = jnp.arange(pad, dtype=jnp.int32) % 128
    src3 = jnp.concatenate([edge_index[0], pad_idx]).reshape(-1, KW, 128)
    dst3 = jnp.concatenate([edge_index[1], N + pad_idx]).reshape(-1, KW, 128)
    comb2d = jnp.concatenate([src3, dst3], axis=1).reshape(-1, 128)

    degp = _sc_degree(comb2d, nwin).reshape(NC, NPAD)

    combT, gateT, gT = _tc_prep(
        _padT(x), _padT(pk_embeddings), _padT(pk_predictions), degp,
        Wg, bg.reshape(F, 1), W1)

    layers = [(b1, ln1_w, ln1_b, W2), (b2, ln2_w, ln2_b, W3),
              (b3, ln3_w, ln3_b, None)]
    auxT = gateT
    hT = None
    for li, (bi, lw, lb, Wn) in enumerate(layers):
        g3 = jnp.transpose(gT.reshape(3, FB, NPAD), (0, 2, 1))
        part = _sc_scatter(g3, comb2d, nwin)
        pT = jnp.transpose(part, (0, 2, 1))
        tT, stats = _tc_post(pT, gT, degp, bi.reshape(F, 1))
        res = _tc_apply(tT, stats, lw.reshape(F, 1), lb.reshape(F, 1),
                        auxT, degp, Wn, gated=(li == 0))
        if Wn is None:
            hT = res[0]
        else:
            hT, gT = res
        auxT = hT

    oT = _tc_heads(hT, combT, Wp1, bp1.reshape(24, 1), Wp2,
                   bp2.reshape(1, 1), Wr1, br1.reshape(24, 1), Wr2,
                   br2.reshape(1, 1), rw.reshape(1, 1))
    return oT[0, :N].reshape(N, 1)


# speedup vs baseline: 18.8080x; 1.0269x over previous
import functools

import jax
import jax.numpy as jnp
from jax import lax
from jax.experimental import pallas as pl
from jax.experimental.pallas import tpu as pltpu
from jax.experimental.pallas import tpu_sc as plsc

N = 100000
F = 48
FB = 16
NC = 2
NS = 16
NW = NC * NS
NPAD = 100352
RPS = NPAD // NS
KW = 6
WEDGE = KW * 128
IB = 2 * KW
BC = 2048
GRID = NPAD // BC
EPS = 1e-5
PREC = lax.Precision.DEFAULT

_mesh = functools.partial(plsc.VectorSubcoreMesh,
                          core_axis_name="c", subcore_axis_name="s")
_sc_params = pltpu.CompilerParams(use_tc_tiling_on_sc=False)


def _zero_slice(zeros_ref, acc_ref, base):
    zn = zeros_ref.shape[0]
    full, rem = RPS // zn, RPS % zn
    for k in range(full):
        pltpu.sync_copy(zeros_ref, acc_ref.at[pl.ds(base + k * zn, zn)])
    if rem:
        pltpu.sync_copy(zeros_ref.at[pl.ds(0, rem)],
                        acc_ref.at[pl.ds(base + full * zn, rem)])


def _sc_degree(comb2d, nwin):
    def body(comb_hbm, out_hbm, ones_v, zeros_v, dst_w, acc):
        c = lax.axis_index("c")
        s = lax.axis_index("s")
        w = c * NS + s
        for i in range(8):
            ones_v[pl.ds(i * 16, 16)] = jnp.ones((16,), jnp.float32)
        for i in range(32):
            zeros_v[pl.ds(i * 16, 16)] = jnp.zeros((16,), jnp.float32)
        _zero_slice(zeros_v, acc, s * RPS)
        plsc.subcore_barrier()

        def win(t, carry):
            rowbase = (w * nwin + t) * IB + KW
            pltpu.sync_copy(comb_hbm.at[pl.ds(rowbase, KW)], dst_w)
            for j in range(KW):
                pltpu.sync_copy(ones_v, acc.at[dst_w.at[j]], add=True)
            return carry

        lax.fori_loop(0, nwin, win, 0)
        plsc.subcore_barrier()
        pltpu.sync_copy(acc.at[pl.ds(s * RPS, RPS)], out_hbm.at[c, s])

    k = pl.kernel(
        body,
        out_type=jax.ShapeDtypeStruct((NC, NS, RPS), jnp.float32),
        mesh=_mesh(),
        compiler_params=_sc_params,
        scratch_types=[
            pltpu.VMEM((128,), jnp.float32),
            pltpu.VMEM((512,), jnp.float32),
            pltpu.VMEM((KW, 128), jnp.int32),
            pltpu.VMEM_SHARED((NPAD,), jnp.float32),
        ],
    )
    return k(comb2d)


def _sc_scatter(g3, comb2d, nwin):
    def body(g3_hbm, comb_hbm,
             out_hbm, zeros_z, idx_w, rows_v, acc, gsem, ssem):
        c = lax.axis_index("c")
        s = lax.axis_index("s")
        w = c * NS + s

        def zb(i, carry):
            zeros_z[i, :] = jnp.zeros((16,), jnp.float32)
            return carry

        lax.fori_loop(0, 128, zb, 0)

        for f in range(3):
            g_hbm = g3_hbm.at[f]
            _zero_slice(zeros_z, acc, s * RPS)
            plsc.subcore_barrier()

            def stage(t, buf):
                pltpu.sync_copy(comb_hbm.at[pl.ds((w * nwin + t) * IB, IB)],
                                idx_w.at[pl.ds(buf * IB, IB)])

            def fire_gathers(buf, g_hbm=g_hbm):
                for j in range(KW):
                    pltpu.async_copy(
                        g_hbm.at[idx_w.at[buf * IB + j]],
                        rows_v.at[pl.ds((buf * KW + j) * 128, 128)], gsem)

            def wait_gathers(buf, g_hbm=g_hbm):
                for j in range(KW):
                    pltpu.make_async_copy(
                        g_hbm.at[idx_w.at[buf * IB + j]],
                        rows_v.at[pl.ds((buf * KW + j) * 128, 128)],
                        gsem).wait()

            def fire_scatters(buf):
                for j in range(KW):
                    pltpu.async_copy(
                        rows_v.at[pl.ds((buf * KW + j) * 128, 128)],
                        acc.at[idx_w.at[buf * IB + KW + j]], ssem, add=True)

            def wait_scatters(buf):
                for j in range(KW):
                    pltpu.make_async_copy(
                        rows_v.at[pl.ds((buf * KW + j) * 128, 128)],
                        acc.at[idx_w.at[buf * IB + KW + j]], ssem).wait()

            stage(0, 0)
            fire_gathers(0)

            def win(t, carry, g_hbm=g_hbm):
                cur = lax.rem(t, 2)
                nxt = 1 - cur

                @pl.when(t > 0)
                def _():
                    wait_scatters(nxt)

                stage(jnp.minimum(t + 1, nwin - 1), nxt)
                wait_gathers(cur)
                fire_gathers(nxt)
                fire_scatters(cur)
                return carry

            lax.fori_loop(0, nwin, win, 0)
            last = (nwin - 1) % 2
            wait_scatters(last)
            wait_gathers(1 - last)
            plsc.subcore_barrier()
            pltpu.sync_copy(
                acc.at[pl.ds(s * RPS, RPS)],
                out_hbm.at[c, pl.ds(s * RPS, RPS), pl.ds(f * FB, FB)])

    k = pl.kernel(
        body,
        out_type=jax.ShapeDtypeStruct((NC, NPAD, F), jnp.float32),
        mesh=_mesh(),
        compiler_params=_sc_params,
        scratch_types=[
            pltpu.VMEM((128, FB), jnp.float32),
            pltpu.VMEM((2 * IB, 128), jnp.int32),
            pltpu.VMEM((2 * WEDGE, FB), jnp.float32),
            pltpu.VMEM_SHARED((NPAD, FB), jnp.float32),
            pltpu.SemaphoreType.DMA,
            pltpu.SemaphoreType.DMA,
        ],
    )
    return k(g3, comb2d)


def _col_spec(rows):
    return pl.BlockSpec((rows, BC), lambda i: (0, i))


_degp_spec = pl.BlockSpec((NC, BC), lambda i: (0, i))
_part_spec = pl.BlockSpec((NC, F, BC), lambda i: (0, 0, i))
_smem_spec = pl.BlockSpec(memory_space=pltpu.SMEM)


def _full_spec(shape):
    nd = len(shape)
    return pl.BlockSpec(shape, lambda i, nd=nd: (0,) * nd)


def _dinv(degp_blk):
    dp = degp_blk
    return lax.rsqrt(1.0 + dp[0:1, :] + dp[1:2, :])


def _prep_body(x_r, pk_r, pkp_r, degp_r, wg_r, bg_r, w1_r,
               comb_o, gate_o, g_o):
    comb = jnp.concatenate([x_r[...], pk_r[...], pkp_r[...]], axis=0)
    comb_o[...] = comb
    gate_o[...] = jax.nn.sigmoid(
        jnp.dot(wg_r[...], comb, precision=PREC) + bg_r[...])
    g_o[...] = jnp.dot(w1_r[...], comb, precision=PREC) * _dinv(degp_r[...])


def _tc_prep(xT, pkT, pkpT, degp, Wg, bgT, W1):
    return pl.pallas_call(
        _prep_body,
        grid=(GRID,),
        in_specs=[_col_spec(32), _col_spec(15), _col_spec(1), _degp_spec,
                  _full_spec((F, F)), _full_spec((F, 1)), _full_spec((F, F))],
        out_specs=(_col_spec(F), _col_spec(F), _col_spec(F)),
        out_shape=(jax.ShapeDtypeStruct((F, NPAD), jnp.float32),
                   jax.ShapeDtypeStruct((F, NPAD), jnp.float32),
                   jax.ShapeDtypeStruct((F, NPAD), jnp.float32)),
    )(xT, pkT, pkpT, degp, Wg, bgT, W1)


def _post_body(p_r, g_r, degp_r, b_r, t_o, st_o):
    i = pl.program_id(0)
    p = p_r[...]
    t = (p[0] + p[1] + g_r[...]) * _dinv(degp_r[...]) + b_r[...]
    t_o[...] = t
    col = lax.broadcasted_iota(jnp.int32, (1, BC), 1)
    tm = jnp.where(col < (N - i * BC), t, 0.0)

    @pl.when(i == 0)
    def _():
        st_o[0, 0] = 0.0
        st_o[0, 1] = 0.0

    st_o[0, 0] += jnp.sum(tm)
    st_o[0, 1] += jnp.sum(tm * tm)


def _tc_post(pT, gT, degp, bT):
    return pl.pallas_call(
        _post_body,
        grid=(GRID,),
        in_specs=[_part_spec, _col_spec(F), _degp_spec, _full_spec((F, 1))],
        out_specs=(_col_spec(F), _smem_spec),
        out_shape=(jax.ShapeDtypeStruct((F, NPAD), jnp.float32),
                   jax.ShapeDtypeStruct((1, 2), jnp.float32)),
    )(pT, gT, degp, bT)


def _apply_body(gated, with_next, t_r, st_r, lw_r, lb_r, aux_r, degp_r,
                *rest):
    inv_cnt = 1.0 / (N * F)
    mean = st_r[0, 0] * inv_cnt
    var = st_r[0, 1] * inv_cnt - mean * mean
    std = jnp.sqrt(jnp.maximum(var, 0.0))
    tn = (t_r[...] - mean) / (std + EPS) * lw_r[...] + lb_r[...]
    r = jnp.maximum(tn, 0.0)
    h = r * aux_r[...] if gated else aux_r[...] + r
    if with_next:
        wn_r, h_o, g_o = rest
        h_o[...] = h
        g_o[...] = jnp.dot(wn_r[...], h, precision=PREC) * _dinv(degp_r[...])
    else:
        (h_o,) = rest
        h_o[...] = h


def _tc_apply(tT, stats, lwT, lbT, auxT, degp, Wn, gated):
    with_next = Wn is not None
    in_specs = [_col_spec(F), _smem_spec, _full_spec((F, 1)),
                _full_spec((F, 1)), _col_spec(F), _degp_spec]
    args = [tT, stats, lwT, lbT, auxT, degp]
    out_specs = [_col_spec(F)]
    out_shape = [jax.ShapeDtypeStruct((F, NPAD), jnp.float32)]
    if with_next:
        in_specs.append(_full_spec((F, F)))
        args.append(Wn)
        out_specs.append(_col_spec(F))
        out_shape.append(jax.ShapeDtypeStruct((F, NPAD), jnp.float32))
    return pl.pallas_call(
        functools.partial(_apply_body, gated, with_next),
        grid=(GRID,),
        in_specs=in_specs,
        out_specs=tuple(out_specs),
        out_shape=tuple(out_shape),
    )(*args)


def _heads_body(h_r, c_r, wp1_r, bp1_r, wp2_r, bp2_r,
                wr1_r, br1_r, wr2_r, br2_r, rw_r, o_r):
    pm = jnp.dot(wp2_r[...], jnp.maximum(
        jnp.dot(wp1_r[...], h_r[...], precision=PREC) + bp1_r[...], 0.0),
        precision=PREC) + bp2_r[0, 0]
    pr = jnp.dot(wr2_r[...], jnp.maximum(
        jnp.dot(wr1_r[...], c_r[...], precision=PREC) + br1_r[...], 0.0),
        precision=PREC) + br2_r[0, 0]
    o_r[...] = pm + rw_r[0, 0] * pr


def _tc_heads(hT, combT, Wp1, bp1T, Wp2, bp2s, Wr1, br1T, Wr2, br2s, rws):
    return pl.pallas_call(
        _heads_body,
        grid=(GRID,),
        in_specs=[_col_spec(F), _col_spec(F),
                  _full_spec((24, F)), _full_spec((24, 1)),
                  _full_spec((1, 24)), _smem_spec,
                  _full_spec((24, F)), _full_spec((24, 1)),
                  _full_spec((1, 24)), _smem_spec, _smem_spec],
        out_specs=_col_spec(1),
        out_shape=jax.ShapeDtypeStruct((1, NPAD), jnp.float32),
    )(hT, combT, Wp1, bp1T, Wp2, bp2s, Wr1, br1T, Wr2, br2s, rws)


def _padT(a):
    return jnp.pad(a.T, ((0, 0), (0, NPAD - a.shape[0])))


def kernel(x, pk_embeddings, pk_predictions, edge_index, Wg, bg, W1, b1,
           W2, b2, W3, b3, ln1_w, ln1_b, ln2_w, ln2_b, ln3_w, ln3_b,
           Wp1, bp1, Wp2, bp2, Wr1, br1, Wr2, br2, rw):
    E = edge_index.shape[1]
    nwin = -(-E // (NW * WEDGE))
    pad = NW * WEDGE * nwin - E
    pad_idx = jnp.arange(pad, dtype=jnp.int32) % 128
    src3 = jnp.concatenate([edge_index[0], pad_idx]).reshape(-1, KW, 128)
    dst3 = jnp.concatenate([edge_index[1], N + pad_idx]).reshape(-1, KW, 128)
    comb2d = jnp.concatenate([src3, dst3], axis=1).reshape(-1, 128)

    degp = _sc_degree(comb2d, nwin).reshape(NC, NPAD)

    combT, gateT, gT = _tc_prep(
        _padT(x), _padT(pk_embeddings), _padT(pk_predictions), degp,
        Wg, bg.reshape(F, 1), W1)

    layers = [(b1, ln1_w, ln1_b, W2), (b2, ln2_w, ln2_b, W3),
              (b3, ln3_w, ln3_b, None)]
    auxT = gateT
    hT = None
    for li, (bi, lw, lb, Wn) in enumerate(layers):
        g3 = jnp.transpose(gT.reshape(3, FB, NPAD), (0, 2, 1))
        part = _sc_scatter(g3, comb2d, nwin)
        pT = jnp.transpose(part, (0, 2, 1))
        tT, stats = _tc_post(pT, gT, degp, bi.reshape(F, 1))
        res = _tc_apply(tT, stats, lw.reshape(F, 1), lb.reshape(F, 1),
                        auxT, degp, Wn, gated=(li == 0))
        if Wn is None:
            hT = res[0]
        else:
            hT, gT = res
        auxT = hT

    oT = _tc_heads(hT, combT, Wp1, bp1.reshape(24, 1), Wp2,
                   bp2.reshape(1, 1), Wr1, br1.reshape(24, 1), Wr2,
                   br2.reshape(1, 1), rw.reshape(1, 1))
    return oT[0, :N].reshape(N, 1)
